# baseline (speedup 1.0000x reference)
import functools

import jax
import jax.numpy as jnp
from jax import lax
from jax.experimental import pallas as pl
from jax.experimental.pallas import tpu as pltpu
from jax.experimental.pallas import tpu_sc as plsc

N_NODES = 2048
N_EDGES = 32768
N_LG_EDGES = 131072
DIM = 128

NC = 2
NS = 16
CHUNK = 128

_SC_MESH = plsc.VectorSubcoreMesh(core_axis_name="c", subcore_axis_name="s")


@functools.cache
def _make_seg_partial(n_tables, n_edges, seg_rows):
    ipt = n_edges // (NC * NS * CHUNK)
    rows_per_tile = seg_rows // NS
    nbuf = 4 // n_tables
    ngroups = ipt // nbuf

    def body(*args):
        tables = args[:n_tables]
        src_hbm, dst_hbm, zeros_hbm, out_hbm = args[n_tables:n_tables + 4]
        src_i, dst_i, rows_v, accum, gsem, ssem = args[n_tables + 4:]
        cid = lax.axis_index("c")
        sid = lax.axis_index("s")
        zr = sid * rows_per_tile
        pltpu.sync_copy(zeros_hbm.at[pl.ds(zr, rows_per_tile)],
                        accum.at[pl.ds(zr, rows_per_tile)])
        row_base = (cid * NS + sid) * ipt
        pltpu.sync_copy(src_hbm.at[pl.ds(row_base, ipt)], src_i)
        pltpu.sync_copy(dst_hbm.at[pl.ds(row_base, ipt)], dst_i)
        plsc.subcore_barrier()

        def group(gi, carry):
            gh = []
            for b in range(nbuf):
                k = gi * nbuf + b
                for t in range(n_tables):
                    gh.append(pltpu.async_copy(
                        tables[t].at[src_i.at[k]],
                        rows_v.at[pl.ds((b * n_tables + t) * CHUNK, CHUNK)],
                        gsem))
            for h in gh:
                h.wait()
            sh = []
            for b in range(nbuf):
                k = gi * nbuf + b
                for t in range(n_tables):
                    sh.append(pltpu.async_copy(
                        rows_v.at[pl.ds((b * n_tables + t) * CHUNK, CHUNK)],
                        accum.at[dst_i.at[k]], ssem, add=True))
            for h in sh:
                h.wait()
            return carry

        lax.fori_loop(0, ngroups, group, 0)
        plsc.subcore_barrier()
        pltpu.sync_copy(accum.at[pl.ds(zr, rows_per_tile)],
                        out_hbm.at[pl.ds(cid * seg_rows + zr, rows_per_tile)])

    return pl.kernel(
        body,
        mesh=_SC_MESH,
        out_type=jax.ShapeDtypeStruct((NC * seg_rows, DIM), jnp.float32),
        scratch_types=[
            pltpu.VMEM((ipt, CHUNK), jnp.int32),
            pltpu.VMEM((ipt, CHUNK), jnp.int32),
            pltpu.VMEM((4 * CHUNK, DIM), jnp.float32),
            pltpu.VMEM_SHARED((seg_rows, DIM), jnp.float32),
            pltpu.SemaphoreType.DMA,
            pltpu.SemaphoreType.DMA,
        ],
    )


LG_RANGE = 8192
LG_ACC_ROWS = 8320
LG_ZERO_PER_TILE = LG_ACC_ROWS // NS
LG_WB_PER_TILE = LG_RANGE // NS


@functools.cache
def _make_seg_ranged(n_edges, seg_rows, nbuf=2, chunk=128):
    ipt = n_edges // (NS * chunk)
    n_ranges = seg_rows // LG_RANGE
    n_ranges_per_sc = n_ranges // NC
    ngroups = ipt // nbuf

    def body(table_hbm, src_hbm, dstall_hbm, zeros_hbm, out_hbm,
             src_i, dst_i, rows_v, accum, gsem, ssem):
        cid = lax.axis_index("c")
        sid = lax.axis_index("s")
        zr = sid * LG_ZERO_PER_TILE
        row_base = sid * ipt
        pltpu.sync_copy(src_hbm.at[pl.ds(row_base, ipt)], src_i)
        for r in range(n_ranges_per_sc):
            rng = cid * n_ranges_per_sc + r
            pltpu.sync_copy(zeros_hbm.at[pl.ds(zr, LG_ZERO_PER_TILE)],
                            accum.at[pl.ds(zr, LG_ZERO_PER_TILE)])
            pltpu.sync_copy(
                dstall_hbm.at[pl.ds(rng * (NS * ipt) + row_base, ipt)], dst_i)
            plsc.subcore_barrier()

            def group(gi, carry):
                gh = []
                for b in range(nbuf):
                    k = gi * nbuf + b
                    gh.append(pltpu.async_copy(
                        table_hbm.at[src_i.at[k]],
                        rows_v.at[pl.ds(b * chunk, chunk)], gsem))
                for h in gh:
                    h.wait()
                sh = []
                for b in range(nbuf):
                    k = gi * nbuf + b
                    sh.append(pltpu.async_copy(
                        rows_v.at[pl.ds(b * chunk, chunk)],
                        accum.at[dst_i.at[k]], ssem, add=True))
                for h in sh:
                    h.wait()
                return carry

            lax.fori_loop(0, ngroups, group, 0)
            plsc.subcore_barrier()
            wb = sid * LG_WB_PER_TILE
            pltpu.sync_copy(accum.at[pl.ds(wb, LG_WB_PER_TILE)],
                            out_hbm.at[pl.ds(rng * LG_RANGE + wb, LG_WB_PER_TILE)])
            plsc.subcore_barrier()

    return pl.kernel(
        body,
        mesh=_SC_MESH,
        out_type=jax.ShapeDtypeStruct((seg_rows, DIM), jnp.float32),
        scratch_types=[
            pltpu.VMEM((ipt, chunk), jnp.int32),
            pltpu.VMEM((ipt, chunk), jnp.int32),
            pltpu.VMEM((nbuf * chunk, DIM), jnp.float32),
            pltpu.VMEM_SHARED((LG_ACC_ROWS, DIM), jnp.float32),
            pltpu.SemaphoreType.DMA,
            pltpu.SemaphoreType.DMA,
        ],
    )


def _dual_mm_body(p_ref, lgx_ref, x_ref, a_ref, b_ref):
    j = pl.program_id(0)
    p = p_ref[...].astype(jnp.bfloat16)
    a_up = jnp.dot(p, lgx_ref[...].astype(jnp.bfloat16),
                   preferred_element_type=jnp.float32)

    @pl.when(j == 0)
    def _init():
        a_ref[...] = jnp.zeros_like(a_ref)

    a_ref[...] += a_up
    b_ref[...] = lax.dot_general(
        p, x_ref[...].astype(jnp.bfloat16), (((0,), (0,)), ((), ())),
        preferred_element_type=jnp.float32)


def _dual_matmul(pm_pd, lg_x, x, block_e=512):
    ne = N_EDGES // block_e
    return pl.pallas_call(
        _dual_mm_body,
        grid=(ne,),
        in_specs=[
            pl.BlockSpec((N_NODES, block_e), lambda j: (0, j)),
            pl.BlockSpec((block_e, DIM), lambda j: (j, 0)),
            pl.BlockSpec((N_NODES, DIM), lambda j: (0, 0)),
        ],
        out_specs=[
            pl.BlockSpec((N_NODES, DIM), lambda j: (0, 0)),
            pl.BlockSpec((block_e, DIM), lambda j: (j, 0)),
        ],
        out_shape=[
            jax.ShapeDtypeStruct((N_NODES, DIM), jnp.float32),
            jax.ShapeDtypeStruct((N_EDGES, DIM), jnp.float32),
        ],
    )(pm_pd, lg_x, x)


def _make_proj_body(split_z):
    def body(*refs):
        if split_z:
            (feat_ref, deg_ref, z1a_ref, z1b_ref, z2a_ref, z2b_ref, fuse_ref,
             wp_ref, wd_ref, w0_ref, w1_ref, wf_ref, bias_ref,
             out_ref, stats_ref) = refs
            z1 = z1a_ref[...] + z1b_ref[...]
            z2 = z2a_ref[...] + z2b_ref[...]
        else:
            (feat_ref, deg_ref, z1_ref, z2_ref, fuse_ref,
             wp_ref, wd_ref, w0_ref, w1_ref, wf_ref, bias_ref,
             out_ref, stats_ref) = refs
            z1 = z1_ref[...]
            z2 = z2_ref[...]
        i = pl.program_id(0)
        feat = feat_ref[...]
        s = jnp.dot(feat, wp_ref[...], preferred_element_type=jnp.float32)
        s = s + jnp.dot(feat * deg_ref[...], wd_ref[...],
                        preferred_element_type=jnp.float32)
        s = s + jnp.dot(z1, w0_ref[...], preferred_element_type=jnp.float32)
        s = s + jnp.dot(z2, w1_ref[...], preferred_element_type=jnp.float32)
        s = s + jnp.dot(fuse_ref[...], wf_ref[...],
                        preferred_element_type=jnp.float32)
        s = s + bias_ref[...]
        col = lax.broadcasted_iota(jnp.int32, s.shape, 1)
        r = jnp.where(col < DIM // 2, s, jnp.maximum(s, 0.0))
        out_ref[...] = r

        @pl.when(i == 0)
        def _init():
            stats_ref[...] = jnp.zeros_like(stats_ref)

        stats_ref[0:1, :] += jnp.sum(r, axis=0, keepdims=True)
        stats_ref[1:2, :] += jnp.sum(r * r, axis=0, keepdims=True)

    return body


def _proj_stats(feat, deg, zs, fuse_in, wp, wd, w0, w1, wf, bias, block_r):
    rows = feat.shape[0]
    ni = rows // block_r
    split_z = len(zs) == 4
    row_spec = pl.BlockSpec((block_r, DIM), lambda i: (i, 0))
    w_spec = pl.BlockSpec((DIM, DIM), lambda i: (0, 0))
    return pl.pallas_call(
        _make_proj_body(split_z),
        grid=(ni,),
        in_specs=[
            row_spec,
            pl.BlockSpec((block_r, 1), lambda i: (i, 0)),
        ] + [row_spec] * (len(zs) + 1) + [
            w_spec, w_spec, w_spec, w_spec, w_spec,
            pl.BlockSpec((1, DIM), lambda i: (0, 0)),
        ],
        out_specs=[
            row_spec,
            pl.BlockSpec((2, DIM), lambda i: (0, 0)),
        ],
        out_shape=[
            jax.ShapeDtypeStruct((rows, DIM), jnp.float32),
            jax.ShapeDtypeStruct((2, DIM), jnp.float32),
        ],
    )(feat, deg, *zs, fuse_in, wp, wd, w0, w1, wf, bias)


def _norm_body(r_ref, stats_ref, gamma_ref, beta_ref, inv_rows_ref, out_ref):
    inv_rows = inv_rows_ref[0]
    mean = stats_ref[0:1, :] * inv_rows
    ex2 = stats_ref[1:2, :] * inv_rows
    var = ex2 - mean * mean
    inv = lax.rsqrt(var + 1e-5)
    out_ref[...] = (r_ref[...] - mean) * inv * gamma_ref[...] + beta_ref[...]


def _normalize(r, stats, gamma, beta, block_r):
    rows = r.shape[0]
    ni = rows // block_r
    inv_rows = jnp.full((1,), 1.0 / rows, jnp.float32)
    row_spec = pl.BlockSpec((block_r, DIM), lambda i: (i, 0))
    vec_spec = pl.BlockSpec((1, DIM), lambda i: (0, 0))
    return pl.pallas_call(
        _norm_body,
        grid=(ni,),
        in_specs=[
            row_spec,
            pl.BlockSpec((2, DIM), lambda i: (0, 0)),
            vec_spec, vec_spec,
            pl.BlockSpec(memory_space=pltpu.SMEM),
        ],
        out_specs=row_spec,
        out_shape=jax.ShapeDtypeStruct((rows, DIM), jnp.float32),
    )(r, stats, gamma.reshape(1, DIM), beta.reshape(1, DIM), inv_rows)


def _core(feat, deg, zs, fuse_in, wp, bp, wd, bd, w0, b0, w1, b1,
          wf, bf, gamma, beta, block_r):
    bias = (bp + bd + b0 + b1 + bf).reshape(1, DIM)
    r, stats = _proj_stats(feat, deg, zs, fuse_in,
                           wp, wd, w0, w1, wf, bias, block_r)
    return _normalize(r, stats, gamma, beta, block_r)


def kernel(x, lg_x, deg_g, deg_lg, pm_pd, edge_index_g, edge_index_lg,
           W_g_prev, b_g_prev, W_g_deg, b_g_deg, W_g_r0, b_g_r0,
           W_g_r1, b_g_r1, W_g_fuse, b_g_fuse, gamma_g, beta_g,
           W_lg_prev, b_lg_prev, W_lg_deg, b_lg_deg, W_lg_r0, b_lg_r0,
           W_lg_r1, b_lg_r1, W_lg_fuse, b_lg_fuse, gamma_lg, beta_lg):
    src_g = edge_index_g[0].reshape(-1, CHUNK)
    dst_g = edge_index_g[1].reshape(-1, CHUNK)
    src_lg = edge_index_lg[0]
    dst_lg = edge_index_lg[1]

    fuse_g_in, fuse_lg_in = _dual_matmul(pm_pd, lg_x, x)

    zeros_g = jnp.zeros((N_NODES, DIM), jnp.float32)
    zeros_lg = jnp.zeros((LG_ACC_ROWS, DIM), jnp.float32)

    clamped = []
    for rng in range(N_EDGES // LG_RANGE):
        loc = dst_lg - rng * LG_RANGE
        ok = (loc >= 0) & (loc < LG_RANGE)
        clamped.append(jnp.where(ok, loc, LG_RANGE).reshape(-1, CHUNK))
    dstall_lg = jnp.concatenate(clamped, axis=0)
    src_lg2 = src_lg.reshape(-1, CHUNK)

    p1 = _make_seg_partial(1, N_EDGES, N_NODES)(x, src_g, dst_g, zeros_g)
    p2 = _make_seg_partial(2, N_EDGES, N_NODES)(
        p1[:N_NODES], p1[N_NODES:], src_g, dst_g, zeros_g)

    lg_kern = _make_seg_ranged(N_LG_EDGES, N_EDGES)

    def _lg_round(tab, _):
        out = lg_kern(tab, src_lg2, dstall_lg, zeros_lg)
        return out, out

    _, zl = lax.scan(_lg_round, lg_x, None, length=2)
    z1l, z2l = zl[0], zl[1]

    out_g = _core(x, deg_g,
                  (p1[:N_NODES], p1[N_NODES:], p2[:N_NODES], p2[N_NODES:]),
                  fuse_g_in,
                  W_g_prev, b_g_prev, W_g_deg, b_g_deg, W_g_r0, b_g_r0,
                  W_g_r1, b_g_r1, W_g_fuse, b_g_fuse, gamma_g, beta_g,
                  block_r=512)
    out_lg = _core(lg_x, deg_lg, (z1l, z2l), fuse_lg_in,
                   W_lg_prev, b_lg_prev, W_lg_deg, b_lg_deg, W_lg_r0, b_lg_r0,
                   W_lg_r1, b_lg_r1, W_lg_fuse, b_lg_fuse, gamma_lg, beta_lg,
                   block_r=512)
    return (out_g, out_lg)


# speedup vs baseline: 1.0003x; 1.0003x over previous
import functools

import jax
import jax.numpy as jnp
from jax import lax
from jax.experimental import pallas as pl
from jax.experimental.pallas import tpu as pltpu
from jax.experimental.pallas import tpu_sc as plsc

N_NODES = 2048
N_EDGES = 32768
N_LG_EDGES = 131072
DIM = 128

NC = 2
NS = 16
CHUNK = 128

_SC_MESH = plsc.VectorSubcoreMesh(core_axis_name="c", subcore_axis_name="s")


@functools.cache
def _make_seg_partial(n_tables, n_edges, seg_rows):
    ipt = n_edges // (NC * NS * CHUNK)
    rows_per_tile = seg_rows // NS
    nbuf = 4 // n_tables
    ngroups = ipt // nbuf

    def body(*args):
        tables = args[:n_tables]
        src_hbm, dst_hbm, zeros_hbm, out_hbm = args[n_tables:n_tables + 4]
        src_i, dst_i, rows_v, accum, gsem, ssem = args[n_tables + 4:]
        cid = lax.axis_index("c")
        sid = lax.axis_index("s")
        zr = sid * rows_per_tile
        pltpu.sync_copy(zeros_hbm.at[pl.ds(zr, rows_per_tile)],
                        accum.at[pl.ds(zr, rows_per_tile)])
        row_base = (cid * NS + sid) * ipt
        pltpu.sync_copy(src_hbm.at[pl.ds(row_base, ipt)], src_i)
        pltpu.sync_copy(dst_hbm.at[pl.ds(row_base, ipt)], dst_i)
        plsc.subcore_barrier()

        def group(gi, carry):
            gh = []
            for b in range(nbuf):
                k = gi * nbuf + b
                for t in range(n_tables):
                    gh.append(pltpu.async_copy(
                        tables[t].at[src_i.at[k]],
                        rows_v.at[pl.ds((b * n_tables + t) * CHUNK, CHUNK)],
                        gsem))
            for h in gh:
                h.wait()
            sh = []
            for b in range(nbuf):
                k = gi * nbuf + b
                for t in range(n_tables):
                    sh.append(pltpu.async_copy(
                        rows_v.at[pl.ds((b * n_tables + t) * CHUNK, CHUNK)],
                        accum.at[dst_i.at[k]], ssem, add=True))
            for h in sh:
                h.wait()
            return carry

        lax.fori_loop(0, ngroups, group, 0)
        plsc.subcore_barrier()
        pltpu.sync_copy(accum.at[pl.ds(zr, rows_per_tile)],
                        out_hbm.at[pl.ds(cid * seg_rows + zr, rows_per_tile)])

    return pl.kernel(
        body,
        mesh=_SC_MESH,
        out_type=jax.ShapeDtypeStruct((NC * seg_rows, DIM), jnp.float32),
        scratch_types=[
            pltpu.VMEM((ipt, CHUNK), jnp.int32),
            pltpu.VMEM((ipt, CHUNK), jnp.int32),
            pltpu.VMEM((4 * CHUNK, DIM), jnp.float32),
            pltpu.VMEM_SHARED((seg_rows, DIM), jnp.float32),
            pltpu.SemaphoreType.DMA,
            pltpu.SemaphoreType.DMA,
        ],
    )


LG_RANGE = 8192
LG_ACC_ROWS = 8320
LG_ZERO_PER_TILE = LG_ACC_ROWS // NS
LG_WB_PER_TILE = LG_RANGE // NS


@functools.cache
def _make_seg_ranged(n_edges, seg_rows, nbuf=2, chunk=128):
    ipt = n_edges // (NS * chunk)
    n_ranges = seg_rows // LG_RANGE
    n_ranges_per_sc = n_ranges // NC
    ngroups = ipt // nbuf

    def body(table_hbm, src_hbm, dstall_hbm, zeros_hbm, out_hbm,
             src_i, dst_i, rows_v, accum, gsem, ssem):
        cid = lax.axis_index("c")
        sid = lax.axis_index("s")
        zr = sid * LG_ZERO_PER_TILE
        row_base = sid * ipt
        pltpu.sync_copy(src_hbm.at[pl.ds(row_base, ipt)], src_i)
        for r in range(n_ranges_per_sc):
            rng = cid * n_ranges_per_sc + r
            pltpu.sync_copy(zeros_hbm.at[pl.ds(zr, LG_ZERO_PER_TILE)],
                            accum.at[pl.ds(zr, LG_ZERO_PER_TILE)])
            pltpu.sync_copy(
                dstall_hbm.at[pl.ds(rng * (NS * ipt) + row_base, ipt)], dst_i)
            plsc.subcore_barrier()

            def group(gi, carry):
                gh = []
                for b in range(nbuf):
                    k = gi * nbuf + b
                    gh.append(pltpu.async_copy(
                        table_hbm.at[src_i.at[k]],
                        rows_v.at[pl.ds(b * chunk, chunk)], gsem))
                for h in gh:
                    h.wait()
                sh = []
                for b in range(nbuf):
                    k = gi * nbuf + b
                    sh.append(pltpu.async_copy(
                        rows_v.at[pl.ds(b * chunk, chunk)],
                        accum.at[dst_i.at[k]], ssem, add=True))
                for h in sh:
                    h.wait()
                return carry

            lax.fori_loop(0, ngroups, group, 0)
            plsc.subcore_barrier()
            wb = sid * LG_WB_PER_TILE
            pltpu.sync_copy(accum.at[pl.ds(wb, LG_WB_PER_TILE)],
                            out_hbm.at[pl.ds(rng * LG_RANGE + wb, LG_WB_PER_TILE)])
            plsc.subcore_barrier()

    return pl.kernel(
        body,
        mesh=_SC_MESH,
        out_type=jax.ShapeDtypeStruct((seg_rows, DIM), jnp.float32),
        scratch_types=[
            pltpu.VMEM((ipt, chunk), jnp.int32),
            pltpu.VMEM((ipt, chunk), jnp.int32),
            pltpu.VMEM((nbuf * chunk, DIM), jnp.float32),
            pltpu.VMEM_SHARED((LG_ACC_ROWS, DIM), jnp.float32),
            pltpu.SemaphoreType.DMA,
            pltpu.SemaphoreType.DMA,
        ],
    )


def _dual_mm_body(p_ref, lgx_ref, x_ref, a_ref, b_ref):
    j = pl.program_id(0)
    p = p_ref[...]
    a_up = jnp.dot(p, lgx_ref[...], preferred_element_type=jnp.float32)

    @pl.when(j == 0)
    def _init():
        a_ref[...] = jnp.zeros_like(a_ref)

    a_ref[...] += a_up
    b_ref[...] = lax.dot_general(
        p, x_ref[...], (((0,), (0,)), ((), ())),
        preferred_element_type=jnp.float32)


def _dual_matmul(pm_pd, lg_x, x, block_e=512):
    ne = N_EDGES // block_e
    return pl.pallas_call(
        _dual_mm_body,
        grid=(ne,),
        in_specs=[
            pl.BlockSpec((N_NODES, block_e), lambda j: (0, j)),
            pl.BlockSpec((block_e, DIM), lambda j: (j, 0)),
            pl.BlockSpec((N_NODES, DIM), lambda j: (0, 0)),
        ],
        out_specs=[
            pl.BlockSpec((N_NODES, DIM), lambda j: (0, 0)),
            pl.BlockSpec((block_e, DIM), lambda j: (j, 0)),
        ],
        out_shape=[
            jax.ShapeDtypeStruct((N_NODES, DIM), jnp.float32),
            jax.ShapeDtypeStruct((N_EDGES, DIM), jnp.float32),
        ],
    )(pm_pd, lg_x, x)


def _make_proj_body(split_z):
    def body(*refs):
        if split_z:
            (feat_ref, deg_ref, z1a_ref, z1b_ref, z2a_ref, z2b_ref, fuse_ref,
             wp_ref, wd_ref, w0_ref, w1_ref, wf_ref, bias_ref,
             out_ref, stats_ref) = refs
            z1 = z1a_ref[...] + z1b_ref[...]
            z2 = z2a_ref[...] + z2b_ref[...]
        else:
            (feat_ref, deg_ref, z1_ref, z2_ref, fuse_ref,
             wp_ref, wd_ref, w0_ref, w1_ref, wf_ref, bias_ref,
             out_ref, stats_ref) = refs
            z1 = z1_ref[...]
            z2 = z2_ref[...]
        i = pl.program_id(0)
        feat = feat_ref[...]
        s = jnp.dot(feat, wp_ref[...], preferred_element_type=jnp.float32)
        s = s + jnp.dot(feat * deg_ref[...], wd_ref[...],
                        preferred_element_type=jnp.float32)
        s = s + jnp.dot(z1, w0_ref[...], preferred_element_type=jnp.float32)
        s = s + jnp.dot(z2, w1_ref[...], preferred_element_type=jnp.float32)
        s = s + jnp.dot(fuse_ref[...], wf_ref[...],
                        preferred_element_type=jnp.float32)
        s = s + bias_ref[...]
        col = lax.broadcasted_iota(jnp.int32, s.shape, 1)
        r = jnp.where(col < DIM // 2, s, jnp.maximum(s, 0.0))
        out_ref[...] = r

        @pl.when(i == 0)
        def _init():
            stats_ref[...] = jnp.zeros_like(stats_ref)

        stats_ref[0:1, :] += jnp.sum(r, axis=0, keepdims=True)
        stats_ref[1:2, :] += jnp.sum(r * r, axis=0, keepdims=True)

    return body


def _proj_stats(feat, deg, zs, fuse_in, wp, wd, w0, w1, wf, bias, block_r):
    rows = feat.shape[0]
    ni = rows // block_r
    split_z = len(zs) == 4
    row_spec = pl.BlockSpec((block_r, DIM), lambda i: (i, 0))
    w_spec = pl.BlockSpec((DIM, DIM), lambda i: (0, 0))
    return pl.pallas_call(
        _make_proj_body(split_z),
        grid=(ni,),
        in_specs=[
            row_spec,
            pl.BlockSpec((block_r, 1), lambda i: (i, 0)),
        ] + [row_spec] * (len(zs) + 1) + [
            w_spec, w_spec, w_spec, w_spec, w_spec,
            pl.BlockSpec((1, DIM), lambda i: (0, 0)),
        ],
        out_specs=[
            row_spec,
            pl.BlockSpec((2, DIM), lambda i: (0, 0)),
        ],
        out_shape=[
            jax.ShapeDtypeStruct((rows, DIM), jnp.float32),
            jax.ShapeDtypeStruct((2, DIM), jnp.float32),
        ],
    )(feat, deg, *zs, fuse_in, wp, wd, w0, w1, wf, bias)


def _norm_body(r_ref, stats_ref, gamma_ref, beta_ref, inv_rows_ref, out_ref):
    inv_rows = inv_rows_ref[0]
    mean = stats_ref[0:1, :] * inv_rows
    ex2 = stats_ref[1:2, :] * inv_rows
    var = ex2 - mean * mean
    inv = lax.rsqrt(var + 1e-5)
    out_ref[...] = (r_ref[...] - mean) * inv * gamma_ref[...] + beta_ref[...]


def _normalize(r, stats, gamma, beta, block_r):
    rows = r.shape[0]
    ni = rows // block_r
    inv_rows = jnp.full((1,), 1.0 / rows, jnp.float32)
    row_spec = pl.BlockSpec((block_r, DIM), lambda i: (i, 0))
    vec_spec = pl.BlockSpec((1, DIM), lambda i: (0, 0))
    return pl.pallas_call(
        _norm_body,
        grid=(ni,),
        in_specs=[
            row_spec,
            pl.BlockSpec((2, DIM), lambda i: (0, 0)),
            vec_spec, vec_spec,
            pl.BlockSpec(memory_space=pltpu.SMEM),
        ],
        out_specs=row_spec,
        out_shape=jax.ShapeDtypeStruct((rows, DIM), jnp.float32),
    )(r, stats, gamma.reshape(1, DIM), beta.reshape(1, DIM), inv_rows)


def _core(feat, deg, zs, fuse_in, wp, bp, wd, bd, w0, b0, w1, b1,
          wf, bf, gamma, beta, block_r):
    bias = (bp + bd + b0 + b1 + bf).reshape(1, DIM)
    r, stats = _proj_stats(feat, deg, zs, fuse_in,
                           wp, wd, w0, w1, wf, bias, block_r)
    return _normalize(r, stats, gamma, beta, block_r)


def kernel(x, lg_x, deg_g, deg_lg, pm_pd, edge_index_g, edge_index_lg,
           W_g_prev, b_g_prev, W_g_deg, b_g_deg, W_g_r0, b_g_r0,
           W_g_r1, b_g_r1, W_g_fuse, b_g_fuse, gamma_g, beta_g,
           W_lg_prev, b_lg_prev, W_lg_deg, b_lg_deg, W_lg_r0, b_lg_r0,
           W_lg_r1, b_lg_r1, W_lg_fuse, b_lg_fuse, gamma_lg, beta_lg):
    src_g = edge_index_g[0].reshape(-1, CHUNK)
    dst_g = edge_index_g[1].reshape(-1, CHUNK)
    src_lg = edge_index_lg[0]
    dst_lg = edge_index_lg[1]

    fuse_g_in, fuse_lg_in = _dual_matmul(pm_pd, lg_x, x)

    zeros_g = jnp.zeros((N_NODES, DIM), jnp.float32)
    zeros_lg = jnp.zeros((LG_ACC_ROWS, DIM), jnp.float32)

    clamped = []
    for rng in range(N_EDGES // LG_RANGE):
        loc = dst_lg - rng * LG_RANGE
        ok = (loc >= 0) & (loc < LG_RANGE)
        clamped.append(jnp.where(ok, loc, LG_RANGE).reshape(-1, CHUNK))
    dstall_lg = jnp.concatenate(clamped, axis=0)
    src_lg2 = src_lg.reshape(-1, CHUNK)

    p1 = _make_seg_partial(1, N_EDGES, N_NODES)(x, src_g, dst_g, zeros_g)
    p2 = _make_seg_partial(2, N_EDGES, N_NODES)(
        p1[:N_NODES], p1[N_NODES:], src_g, dst_g, zeros_g)

    lg_kern = _make_seg_ranged(N_LG_EDGES, N_EDGES)

    def _lg_round(tab, _):
        out = lg_kern(tab, src_lg2, dstall_lg, zeros_lg)
        return out, out

    _, zl = lax.scan(_lg_round, lg_x, None, length=2)
    z1l, z2l = zl[0], zl[1]

    out_g = _core(x, deg_g,
                  (p1[:N_NODES], p1[N_NODES:], p2[:N_NODES], p2[N_NODES:]),
                  fuse_g_in,
                  W_g_prev, b_g_prev, W_g_deg, b_g_deg, W_g_r0, b_g_r0,
                  W_g_r1, b_g_r1, W_g_fuse, b_g_fuse, gamma_g, beta_g,
                  block_r=512)
    out_lg = _core(lg_x, deg_lg, (z1l, z2l), fuse_lg_in,
                   W_lg_prev, b_lg_prev, W_lg_deg, b_lg_deg, W_lg_r0, b_lg_r0,
                   W_lg_r1, b_lg_r1, W_lg_fuse, b_lg_fuse, gamma_lg, beta_lg,
                   block_r=512)
    return (out_g, out_lg)


# speedup vs baseline: 1.0102x; 1.0099x over previous
import functools

import jax
import jax.numpy as jnp
from jax import lax
from jax.experimental import pallas as pl
from jax.experimental.pallas import tpu as pltpu
from jax.experimental.pallas import tpu_sc as plsc

N_NODES = 2048
N_EDGES = 32768
N_LG_EDGES = 131072
DIM = 128

NC = 2
NS = 16
CHUNK = 128

_SC_MESH = plsc.VectorSubcoreMesh(core_axis_name="c", subcore_axis_name="s")


@functools.cache
def _make_seg_partial(n_tables, n_edges, seg_rows):
    ipt = n_edges // (NC * NS * CHUNK)
    rows_per_tile = seg_rows // NS
    nbuf = 4 // n_tables
    ngroups = ipt // nbuf

    def body(*args):
        tables = args[:n_tables]
        src_hbm, dst_hbm, zeros_hbm, out_hbm = args[n_tables:n_tables + 4]
        (src_i, dst_i, rows_v, accum,
         gsem0, gsem1, gsem2, gsem3, ssem) = args[n_tables + 4:]
        gsems = (gsem0, gsem1, gsem2, gsem3)
        cid = lax.axis_index("c")
        sid = lax.axis_index("s")
        zr = sid * rows_per_tile
        pltpu.sync_copy(zeros_hbm.at[pl.ds(zr, rows_per_tile)],
                        accum.at[pl.ds(zr, rows_per_tile)])
        row_base = (cid * NS + sid) * ipt
        pltpu.sync_copy(src_hbm.at[pl.ds(row_base, ipt)], src_i)
        pltpu.sync_copy(dst_hbm.at[pl.ds(row_base, ipt)], dst_i)
        plsc.subcore_barrier()

        def group(gi, carry):
            gh = []
            for b in range(nbuf):
                k = gi * nbuf + b
                for t in range(n_tables):
                    s = b * n_tables + t
                    gh.append(pltpu.async_copy(
                        tables[t].at[src_i.at[k]],
                        rows_v.at[pl.ds(s * CHUNK, CHUNK)],
                        gsems[s]))
            sh = []
            for b in range(nbuf):
                k = gi * nbuf + b
                for t in range(n_tables):
                    s = b * n_tables + t
                    gh[s].wait()
                    sh.append(pltpu.async_copy(
                        rows_v.at[pl.ds(s * CHUNK, CHUNK)],
                        accum.at[dst_i.at[k]], ssem, add=True))
            for h in sh:
                h.wait()
            return carry

        lax.fori_loop(0, ngroups, group, 0)
        plsc.subcore_barrier()
        pltpu.sync_copy(accum.at[pl.ds(zr, rows_per_tile)],
                        out_hbm.at[pl.ds(cid * seg_rows + zr, rows_per_tile)])

    return pl.kernel(
        body,
        mesh=_SC_MESH,
        out_type=jax.ShapeDtypeStruct((NC * seg_rows, DIM), jnp.float32),
        scratch_types=[
            pltpu.VMEM((ipt, CHUNK), jnp.int32),
            pltpu.VMEM((ipt, CHUNK), jnp.int32),
            pltpu.VMEM((4 * CHUNK, DIM), jnp.float32),
            pltpu.VMEM_SHARED((seg_rows, DIM), jnp.float32),
            pltpu.SemaphoreType.DMA,
            pltpu.SemaphoreType.DMA,
            pltpu.SemaphoreType.DMA,
            pltpu.SemaphoreType.DMA,
            pltpu.SemaphoreType.DMA,
        ],
    )


LG_RANGE = 8192
LG_ACC_ROWS = 8320
LG_ZERO_PER_TILE = LG_ACC_ROWS // NS
LG_WB_PER_TILE = LG_RANGE // NS


@functools.cache
def _make_seg_ranged(n_edges, seg_rows, nbuf=2, chunk=128):
    ipt = n_edges // (NS * chunk)
    n_ranges = seg_rows // LG_RANGE
    n_ranges_per_sc = n_ranges // NC
    ngroups = ipt // nbuf

    def body(table_hbm, src_hbm, dstall_hbm, zeros_hbm, out_hbm,
             src_i, dst_i, rows_v, accum, gsem0, gsem1, ssem):
        gsems = (gsem0, gsem1)
        cid = lax.axis_index("c")
        sid = lax.axis_index("s")
        zr = sid * LG_ZERO_PER_TILE
        row_base = sid * ipt
        pltpu.sync_copy(src_hbm.at[pl.ds(row_base, ipt)], src_i)
        for r in range(n_ranges_per_sc):
            rng = cid * n_ranges_per_sc + r
            pltpu.sync_copy(zeros_hbm.at[pl.ds(zr, LG_ZERO_PER_TILE)],
                            accum.at[pl.ds(zr, LG_ZERO_PER_TILE)])
            pltpu.sync_copy(
                dstall_hbm.at[pl.ds(rng * (NS * ipt) + row_base, ipt)], dst_i)
            plsc.subcore_barrier()

            def group(gi, carry):
                gh = []
                for b in range(nbuf):
                    k = gi * nbuf + b
                    gh.append(pltpu.async_copy(
                        table_hbm.at[src_i.at[k]],
                        rows_v.at[pl.ds(b * chunk, chunk)], gsems[b]))
                sh = []
                for b in range(nbuf):
                    k = gi * nbuf + b
                    gh[b].wait()
                    sh.append(pltpu.async_copy(
                        rows_v.at[pl.ds(b * chunk, chunk)],
                        accum.at[dst_i.at[k]], ssem, add=True))
                for h in sh:
                    h.wait()
                return carry

            lax.fori_loop(0, ngroups, group, 0)
            plsc.subcore_barrier()
            wb = sid * LG_WB_PER_TILE
            pltpu.sync_copy(accum.at[pl.ds(wb, LG_WB_PER_TILE)],
                            out_hbm.at[pl.ds(rng * LG_RANGE + wb, LG_WB_PER_TILE)])
            plsc.subcore_barrier()

    return pl.kernel(
        body,
        mesh=_SC_MESH,
        out_type=jax.ShapeDtypeStruct((seg_rows, DIM), jnp.float32),
        scratch_types=[
            pltpu.VMEM((ipt, chunk), jnp.int32),
            pltpu.VMEM((ipt, chunk), jnp.int32),
            pltpu.VMEM((nbuf * chunk, DIM), jnp.float32),
            pltpu.VMEM_SHARED((LG_ACC_ROWS, DIM), jnp.float32),
            pltpu.SemaphoreType.DMA,
            pltpu.SemaphoreType.DMA,
            pltpu.SemaphoreType.DMA,
        ],
    )


def _dual_mm_body(p_ref, lgx_ref, x_ref, a_ref, b_ref):
    j = pl.program_id(0)
    p = p_ref[...]
    a_up = jnp.dot(p, lgx_ref[...], preferred_element_type=jnp.float32)

    @pl.when(j == 0)
    def _init():
        a_ref[...] = jnp.zeros_like(a_ref)

    a_ref[...] += a_up
    b_ref[...] = lax.dot_general(
        p, x_ref[...], (((0,), (0,)), ((), ())),
        preferred_element_type=jnp.float32)


def _dual_matmul(pm_pd, lg_x, x, block_e=512):
    ne = N_EDGES // block_e
    return pl.pallas_call(
        _dual_mm_body,
        grid=(ne,),
        in_specs=[
            pl.BlockSpec((N_NODES, block_e), lambda j: (0, j)),
            pl.BlockSpec((block_e, DIM), lambda j: (j, 0)),
            pl.BlockSpec((N_NODES, DIM), lambda j: (0, 0)),
        ],
        out_specs=[
            pl.BlockSpec((N_NODES, DIM), lambda j: (0, 0)),
            pl.BlockSpec((block_e, DIM), lambda j: (j, 0)),
        ],
        out_shape=[
            jax.ShapeDtypeStruct((N_NODES, DIM), jnp.float32),
            jax.ShapeDtypeStruct((N_EDGES, DIM), jnp.float32),
        ],
    )(pm_pd, lg_x, x)


def _make_proj_body(split_z):
    def body(*refs):
        if split_z:
            (feat_ref, deg_ref, z1a_ref, z1b_ref, z2a_ref, z2b_ref, fuse_ref,
             wp_ref, wd_ref, w0_ref, w1_ref, wf_ref, bias_ref,
             out_ref, stats_ref) = refs
            z1 = z1a_ref[...] + z1b_ref[...]
            z2 = z2a_ref[...] + z2b_ref[...]
        else:
            (feat_ref, deg_ref, z1_ref, z2_ref, fuse_ref,
             wp_ref, wd_ref, w0_ref, w1_ref, wf_ref, bias_ref,
             out_ref, stats_ref) = refs
            z1 = z1_ref[...]
            z2 = z2_ref[...]
        i = pl.program_id(0)
        feat = feat_ref[...]
        s = jnp.dot(feat, wp_ref[...], preferred_element_type=jnp.float32)
        s = s + jnp.dot(feat * deg_ref[...], wd_ref[...],
                        preferred_element_type=jnp.float32)
        s = s + jnp.dot(z1, w0_ref[...], preferred_element_type=jnp.float32)
        s = s + jnp.dot(z2, w1_ref[...], preferred_element_type=jnp.float32)
        s = s + jnp.dot(fuse_ref[...], wf_ref[...],
                        preferred_element_type=jnp.float32)
        s = s + bias_ref[...]
        col = lax.broadcasted_iota(jnp.int32, s.shape, 1)
        r = jnp.where(col < DIM // 2, s, jnp.maximum(s, 0.0))
        out_ref[...] = r

        @pl.when(i == 0)
        def _init():
            stats_ref[...] = jnp.zeros_like(stats_ref)

        stats_ref[0:1, :] += jnp.sum(r, axis=0, keepdims=True)
        stats_ref[1:2, :] += jnp.sum(r * r, axis=0, keepdims=True)

    return body


def _proj_stats(feat, deg, zs, fuse_in, wp, wd, w0, w1, wf, bias, block_r):
    rows = feat.shape[0]
    ni = rows // block_r
    split_z = len(zs) == 4
    row_spec = pl.BlockSpec((block_r, DIM), lambda i: (i, 0))
    w_spec = pl.BlockSpec((DIM, DIM), lambda i: (0, 0))
    return pl.pallas_call(
        _make_proj_body(split_z),
        grid=(ni,),
        in_specs=[
            row_spec,
            pl.BlockSpec((block_r, 1), lambda i: (i, 0)),
        ] + [row_spec] * (len(zs) + 1) + [
            w_spec, w_spec, w_spec, w_spec, w_spec,
            pl.BlockSpec((1, DIM), lambda i: (0, 0)),
        ],
        out_specs=[
            row_spec,
            pl.BlockSpec((2, DIM), lambda i: (0, 0)),
        ],
        out_shape=[
            jax.ShapeDtypeStruct((rows, DIM), jnp.float32),
            jax.ShapeDtypeStruct((2, DIM), jnp.float32),
        ],
    )(feat, deg, *zs, fuse_in, wp, wd, w0, w1, wf, bias)


def _norm_body(r_ref, stats_ref, gamma_ref, beta_ref, inv_rows_ref, out_ref):
    inv_rows = inv_rows_ref[0]
    mean = stats_ref[0:1, :] * inv_rows
    ex2 = stats_ref[1:2, :] * inv_rows
    var = ex2 - mean * mean
    inv = lax.rsqrt(var + 1e-5)
    out_ref[...] = (r_ref[...] - mean) * inv * gamma_ref[...] + beta_ref[...]


def _normalize(r, stats, gamma, beta, block_r):
    rows = r.shape[0]
    ni = rows // block_r
    inv_rows = jnp.full((1,), 1.0 / rows, jnp.float32)
    row_spec = pl.BlockSpec((block_r, DIM), lambda i: (i, 0))
    vec_spec = pl.BlockSpec((1, DIM), lambda i: (0, 0))
    return pl.pallas_call(
        _norm_body,
        grid=(ni,),
        in_specs=[
            row_spec,
            pl.BlockSpec((2, DIM), lambda i: (0, 0)),
            vec_spec, vec_spec,
            pl.BlockSpec(memory_space=pltpu.SMEM),
        ],
        out_specs=row_spec,
        out_shape=jax.ShapeDtypeStruct((rows, DIM), jnp.float32),
    )(r, stats, gamma.reshape(1, DIM), beta.reshape(1, DIM), inv_rows)


def _core(feat, deg, zs, fuse_in, wp, bp, wd, bd, w0, b0, w1, b1,
          wf, bf, gamma, beta, block_r):
    bias = (bp + bd + b0 + b1 + bf).reshape(1, DIM)
    r, stats = _proj_stats(feat, deg, zs, fuse_in,
                           wp, wd, w0, w1, wf, bias, block_r)
    return _normalize(r, stats, gamma, beta, block_r)


def kernel(x, lg_x, deg_g, deg_lg, pm_pd, edge_index_g, edge_index_lg,
           W_g_prev, b_g_prev, W_g_deg, b_g_deg, W_g_r0, b_g_r0,
           W_g_r1, b_g_r1, W_g_fuse, b_g_fuse, gamma_g, beta_g,
           W_lg_prev, b_lg_prev, W_lg_deg, b_lg_deg, W_lg_r0, b_lg_r0,
           W_lg_r1, b_lg_r1, W_lg_fuse, b_lg_fuse, gamma_lg, beta_lg):
    src_g = edge_index_g[0].reshape(-1, CHUNK)
    dst_g = edge_index_g[1].reshape(-1, CHUNK)
    src_lg = edge_index_lg[0]
    dst_lg = edge_index_lg[1]

    fuse_g_in, fuse_lg_in = _dual_matmul(pm_pd, lg_x, x)

    zeros_g = jnp.zeros((N_NODES, DIM), jnp.float32)
    zeros_lg = jnp.zeros((LG_ACC_ROWS, DIM), jnp.float32)

    clamped = []
    for rng in range(N_EDGES // LG_RANGE):
        loc = dst_lg - rng * LG_RANGE
        ok = (loc >= 0) & (loc < LG_RANGE)
        clamped.append(jnp.where(ok, loc, LG_RANGE).reshape(-1, CHUNK))
    dstall_lg = jnp.concatenate(clamped, axis=0)
    src_lg2 = src_lg.reshape(-1, CHUNK)

    p1 = _make_seg_partial(1, N_EDGES, N_NODES)(x, src_g, dst_g, zeros_g)
    p2 = _make_seg_partial(2, N_EDGES, N_NODES)(
        p1[:N_NODES], p1[N_NODES:], src_g, dst_g, zeros_g)

    lg_kern = _make_seg_ranged(N_LG_EDGES, N_EDGES)

    def _lg_round(tab, _):
        out = lg_kern(tab, src_lg2, dstall_lg, zeros_lg)
        return out, out

    _, zl = lax.scan(_lg_round, lg_x, None, length=2)
    z1l, z2l = zl[0], zl[1]

    out_g = _core(x, deg_g,
                  (p1[:N_NODES], p1[N_NODES:], p2[:N_NODES], p2[N_NODES:]),
                  fuse_g_in,
                  W_g_prev, b_g_prev, W_g_deg, b_g_deg, W_g_r0, b_g_r0,
                  W_g_r1, b_g_r1, W_g_fuse, b_g_fuse, gamma_g, beta_g,
                  block_r=512)
    out_lg = _core(lg_x, deg_lg, (z1l, z2l), fuse_lg_in,
                   W_lg_prev, b_lg_prev, W_lg_deg, b_lg_deg, W_lg_r0, b_lg_r0,
                   W_lg_r1, b_lg_r1, W_lg_fuse, b_lg_fuse, gamma_lg, beta_lg,
                   block_r=512)
    return (out_g, out_lg)


# speedup vs baseline: 1.0335x; 1.0230x over previous
import functools

import jax
import jax.numpy as jnp
from jax import lax
from jax.experimental import pallas as pl
from jax.experimental.pallas import tpu as pltpu
from jax.experimental.pallas import tpu_sc as plsc

N_NODES = 2048
N_EDGES = 32768
N_LG_EDGES = 131072
DIM = 128

NC = 2
NS = 16
CHUNK = 128

_SC_MESH = plsc.VectorSubcoreMesh(core_axis_name="c", subcore_axis_name="s")


@functools.cache
def _make_seg_partial(n_tables, n_edges, seg_rows):
    ipt = n_edges // (NC * NS * CHUNK)
    rows_per_tile = seg_rows // NS
    nbuf = 4 // n_tables
    ngroups = ipt // nbuf

    def body(*args):
        tables = args[:n_tables]
        src_hbm, dst_hbm, zeros_hbm, out_hbm = args[n_tables:n_tables + 4]
        (src_i, dst_i, rows_v, accum,
         gsem0, gsem1, gsem2, gsem3, ssem) = args[n_tables + 4:]
        gsems = (gsem0, gsem1, gsem2, gsem3)
        cid = lax.axis_index("c")
        sid = lax.axis_index("s")
        zr = sid * rows_per_tile
        pltpu.sync_copy(zeros_hbm.at[pl.ds(zr, rows_per_tile)],
                        accum.at[pl.ds(zr, rows_per_tile)])
        row_base = (cid * NS + sid) * ipt
        pltpu.sync_copy(src_hbm.at[pl.ds(row_base, ipt)], src_i)
        pltpu.sync_copy(dst_hbm.at[pl.ds(row_base, ipt)], dst_i)
        plsc.subcore_barrier()

        def group(gi, carry):
            gh = []
            for b in range(nbuf):
                k = gi * nbuf + b
                for t in range(n_tables):
                    s = b * n_tables + t
                    gh.append(pltpu.async_copy(
                        tables[t].at[src_i.at[k]],
                        rows_v.at[pl.ds(s * CHUNK, CHUNK)],
                        gsems[s]))
            sh = []
            for b in range(nbuf):
                k = gi * nbuf + b
                for t in range(n_tables):
                    s = b * n_tables + t
                    gh[s].wait()
                    sh.append(pltpu.async_copy(
                        rows_v.at[pl.ds(s * CHUNK, CHUNK)],
                        accum.at[dst_i.at[k]], ssem, add=True))
            for h in sh:
                h.wait()
            return carry

        lax.fori_loop(0, ngroups, group, 0)
        plsc.subcore_barrier()
        pltpu.sync_copy(accum.at[pl.ds(zr, rows_per_tile)],
                        out_hbm.at[pl.ds(cid * seg_rows + zr, rows_per_tile)])

    return pl.kernel(
        body,
        mesh=_SC_MESH,
        out_type=jax.ShapeDtypeStruct((NC * seg_rows, DIM), jnp.float32),
        scratch_types=[
            pltpu.VMEM((ipt, CHUNK), jnp.int32),
            pltpu.VMEM((ipt, CHUNK), jnp.int32),
            pltpu.VMEM((4 * CHUNK, DIM), jnp.float32),
            pltpu.VMEM_SHARED((seg_rows, DIM), jnp.float32),
            pltpu.SemaphoreType.DMA,
            pltpu.SemaphoreType.DMA,
            pltpu.SemaphoreType.DMA,
            pltpu.SemaphoreType.DMA,
            pltpu.SemaphoreType.DMA,
        ],
    )


LG_RANGE = 8192
LG_ACC_ROWS = 8320
LG_ZERO_PER_TILE = LG_ACC_ROWS // NS
LG_WB_PER_TILE = LG_RANGE // NS


@functools.cache
def _make_seg_ranged(n_edges, seg_rows, nbuf=2, chunk=128):
    ipt = n_edges // (NS * chunk)
    n_ranges = seg_rows // LG_RANGE
    n_ranges_per_sc = n_ranges // NC
    ngroups = ipt // nbuf

    def body(table_hbm, src_hbm, dstall_hbm, zeros_hbm, out_hbm,
             src_i, dst_i, rows_v, accum, gsem0, gsem1, ssem):
        gsems = (gsem0, gsem1)
        cid = lax.axis_index("c")
        sid = lax.axis_index("s")
        zr = sid * LG_ZERO_PER_TILE
        row_base = sid * ipt
        pltpu.sync_copy(src_hbm.at[pl.ds(row_base, ipt)], src_i)
        for r in range(n_ranges_per_sc):
            rng = cid * n_ranges_per_sc + r
            pltpu.sync_copy(zeros_hbm.at[pl.ds(zr, LG_ZERO_PER_TILE)],
                            accum.at[pl.ds(zr, LG_ZERO_PER_TILE)])
            pltpu.sync_copy(
                dstall_hbm.at[pl.ds(rng * (NS * ipt) + row_base, ipt)], dst_i)
            plsc.subcore_barrier()

            def group(gi, carry):
                gh = []
                for b in range(nbuf):
                    k = gi * nbuf + b
                    gh.append(pltpu.async_copy(
                        table_hbm.at[src_i.at[k]],
                        rows_v.at[pl.ds(b * chunk, chunk)], gsems[b]))
                sh = []
                for b in range(nbuf):
                    k = gi * nbuf + b
                    gh[b].wait()
                    sh.append(pltpu.async_copy(
                        rows_v.at[pl.ds(b * chunk, chunk)],
                        accum.at[dst_i.at[k]], ssem, add=True))
                for h in sh:
                    h.wait()
                return carry

            lax.fori_loop(0, ngroups, group, 0)
            plsc.subcore_barrier()
            wb = sid * LG_WB_PER_TILE
            pltpu.sync_copy(accum.at[pl.ds(wb, LG_WB_PER_TILE)],
                            out_hbm.at[pl.ds(rng * LG_RANGE + wb, LG_WB_PER_TILE)])
            plsc.subcore_barrier()

    return pl.kernel(
        body,
        mesh=_SC_MESH,
        out_type=jax.ShapeDtypeStruct((seg_rows, DIM), jnp.float32),
        scratch_types=[
            pltpu.VMEM((ipt, chunk), jnp.int32),
            pltpu.VMEM((ipt, chunk), jnp.int32),
            pltpu.VMEM((nbuf * chunk, DIM), jnp.float32),
            pltpu.VMEM_SHARED((LG_ACC_ROWS, DIM), jnp.float32),
            pltpu.SemaphoreType.DMA,
            pltpu.SemaphoreType.DMA,
            pltpu.SemaphoreType.DMA,
        ],
    )


def _dual_mm_body(p_ref, lgx_ref, x_ref, a_ref, b_ref):
    j = pl.program_id(0)
    p = p_ref[...]
    a_up = jnp.dot(p, lgx_ref[...], preferred_element_type=jnp.float32)

    @pl.when(j == 0)
    def _init():
        a_ref[...] = jnp.zeros_like(a_ref)

    a_ref[...] += a_up
    b_ref[...] = lax.dot_general(
        p, x_ref[...], (((0,), (0,)), ((), ())),
        preferred_element_type=jnp.float32)


def _dual_matmul(pm_pd, lg_x, x, block_e=512):
    ne = N_EDGES // block_e
    return pl.pallas_call(
        _dual_mm_body,
        grid=(ne,),
        in_specs=[
            pl.BlockSpec((N_NODES, block_e), lambda j: (0, j)),
            pl.BlockSpec((block_e, DIM), lambda j: (j, 0)),
            pl.BlockSpec((N_NODES, DIM), lambda j: (0, 0)),
        ],
        out_specs=[
            pl.BlockSpec((N_NODES, DIM), lambda j: (0, 0)),
            pl.BlockSpec((block_e, DIM), lambda j: (j, 0)),
        ],
        out_shape=[
            jax.ShapeDtypeStruct((N_NODES, DIM), jnp.float32),
            jax.ShapeDtypeStruct((N_EDGES, DIM), jnp.float32),
        ],
    )(pm_pd, lg_x, x)


def _make_core_body(split_z, ni, block_r, inv_rows):
    def body(*refs):
        if split_z:
            (feat_ref, deg_ref, z1a_ref, z1b_ref, z2a_ref, z2b_ref, fuse_ref,
             wp_ref, wd_ref, w0_ref, w1_ref, wf_ref, bias_ref,
             gamma_ref, beta_ref, out_ref, r_scr, stats_scr) = refs
        else:
            (feat_ref, deg_ref, z1_ref, z2_ref, fuse_ref,
             wp_ref, wd_ref, w0_ref, w1_ref, wf_ref, bias_ref,
             gamma_ref, beta_ref, out_ref, r_scr, stats_scr) = refs
        i = pl.program_id(0)

        @pl.when(i < ni)
        def _proj():
            if split_z:
                z1 = z1a_ref[...] + z1b_ref[...]
                z2 = z2a_ref[...] + z2b_ref[...]
            else:
                z1 = z1_ref[...]
                z2 = z2_ref[...]
            feat = feat_ref[...]
            s = jnp.dot(feat, wp_ref[...], preferred_element_type=jnp.float32)
            s = s + jnp.dot(feat * deg_ref[...], wd_ref[...],
                            preferred_element_type=jnp.float32)
            s = s + jnp.dot(z1, w0_ref[...], preferred_element_type=jnp.float32)
            s = s + jnp.dot(z2, w1_ref[...], preferred_element_type=jnp.float32)
            s = s + jnp.dot(fuse_ref[...], wf_ref[...],
                            preferred_element_type=jnp.float32)
            s = s + bias_ref[...]
            col = lax.broadcasted_iota(jnp.int32, s.shape, 1)
            r = jnp.where(col < DIM // 2, s, jnp.maximum(s, 0.0))
            r_scr[pl.ds(i * block_r, block_r), :] = r

            @pl.when(i == 0)
            def _init():
                stats_scr[...] = jnp.zeros_like(stats_scr)

            stats_scr[0:1, :] += jnp.sum(r, axis=0, keepdims=True)
            stats_scr[1:2, :] += jnp.sum(r * r, axis=0, keepdims=True)

        @pl.when(i >= ni)
        def _norm():
            j = i - ni
            mean = stats_scr[0:1, :] * inv_rows
            ex2 = stats_scr[1:2, :] * inv_rows
            var = ex2 - mean * mean
            inv = lax.rsqrt(var + 1e-5)
            r = r_scr[pl.ds(j * block_r, block_r), :]
            out_ref[...] = (r - mean) * inv * gamma_ref[...] + beta_ref[...]

    return body


def _core(feat, deg, zs, fuse_in, wp, bp, wd, bd, w0, b0, w1, b1,
          wf, bf, gamma, beta, block_r):
    bias = (bp + bd + b0 + b1 + bf).reshape(1, DIM)
    rows = feat.shape[0]
    ni = rows // block_r
    split_z = len(zs) == 4
    last = ni - 1
    row_spec = pl.BlockSpec((block_r, DIM),
                            lambda i: (jnp.minimum(i, last), 0))
    w_spec = pl.BlockSpec((DIM, DIM), lambda i: (0, 0))
    vec_spec = pl.BlockSpec((1, DIM), lambda i: (0, 0))
    return pl.pallas_call(
        _make_core_body(split_z, ni, block_r, 1.0 / rows),
        grid=(2 * ni,),
        in_specs=[
            row_spec,
            pl.BlockSpec((block_r, 1), lambda i: (jnp.minimum(i, last), 0)),
        ] + [row_spec] * (len(zs) + 1) + [
            w_spec, w_spec, w_spec, w_spec, w_spec,
            vec_spec, vec_spec, vec_spec,
        ],
        out_specs=pl.BlockSpec((block_r, DIM),
                               lambda i: (jnp.maximum(i - ni, 0), 0)),
        out_shape=jax.ShapeDtypeStruct((rows, DIM), jnp.float32),
        scratch_shapes=[
            pltpu.VMEM((rows, DIM), jnp.float32),
            pltpu.VMEM((2, DIM), jnp.float32),
        ],
    )(feat, deg, *zs, fuse_in, wp, wd, w0, w1, wf, bias,
      gamma.reshape(1, DIM), beta.reshape(1, DIM))


def kernel(x, lg_x, deg_g, deg_lg, pm_pd, edge_index_g, edge_index_lg,
           W_g_prev, b_g_prev, W_g_deg, b_g_deg, W_g_r0, b_g_r0,
           W_g_r1, b_g_r1, W_g_fuse, b_g_fuse, gamma_g, beta_g,
           W_lg_prev, b_lg_prev, W_lg_deg, b_lg_deg, W_lg_r0, b_lg_r0,
           W_lg_r1, b_lg_r1, W_lg_fuse, b_lg_fuse, gamma_lg, beta_lg):
    src_g = edge_index_g[0].reshape(-1, CHUNK)
    dst_g = edge_index_g[1].reshape(-1, CHUNK)
    src_lg = edge_index_lg[0]
    dst_lg = edge_index_lg[1]

    fuse_g_in, fuse_lg_in = _dual_matmul(pm_pd, lg_x, x)

    zeros_g = jnp.zeros((N_NODES, DIM), jnp.float32)
    zeros_lg = jnp.zeros((LG_ACC_ROWS, DIM), jnp.float32)

    clamped = []
    for rng in range(N_EDGES // LG_RANGE):
        loc = dst_lg - rng * LG_RANGE
        ok = (loc >= 0) & (loc < LG_RANGE)
        clamped.append(jnp.where(ok, loc, LG_RANGE).reshape(-1, CHUNK))
    dstall_lg = jnp.concatenate(clamped, axis=0)
    src_lg2 = src_lg.reshape(-1, CHUNK)

    p1 = _make_seg_partial(1, N_EDGES, N_NODES)(x, src_g, dst_g, zeros_g)
    p2 = _make_seg_partial(2, N_EDGES, N_NODES)(
        p1[:N_NODES], p1[N_NODES:], src_g, dst_g, zeros_g)

    lg_kern = _make_seg_ranged(N_LG_EDGES, N_EDGES)

    def _lg_round(tab, _):
        out = lg_kern(tab, src_lg2, dstall_lg, zeros_lg)
        return out, out

    _, zl = lax.scan(_lg_round, lg_x, None, length=2)
    z1l, z2l = zl[0], zl[1]

    out_g = _core(x, deg_g,
                  (p1[:N_NODES], p1[N_NODES:], p2[:N_NODES], p2[N_NODES:]),
                  fuse_g_in,
                  W_g_prev, b_g_prev, W_g_deg, b_g_deg, W_g_r0, b_g_r0,
                  W_g_r1, b_g_r1, W_g_fuse, b_g_fuse, gamma_g, beta_g,
                  block_r=512)
    out_lg = _core(lg_x, deg_lg, (z1l, z2l), fuse_lg_in,
                   W_lg_prev, b_lg_prev, W_lg_deg, b_lg_deg, W_lg_r0, b_lg_r0,
                   W_lg_r1, b_lg_r1, W_lg_fuse, b_lg_fuse, gamma_lg, beta_lg,
                   block_r=512)
    return (out_g, out_lg)
